# baseline (speedup 1.0000x reference)
import functools

import jax
import jax.numpy as jnp
from jax import lax
from jax.experimental import pallas as pl
from jax.experimental.pallas import tpu as pltpu
from jax.experimental.pallas import tpu_sc as plsc

EMB = 64
NC, NS = 2, 16
NW = NC * NS
CHUNK = 128


def _sc_gather3(tw, cw, ng, temb, cemb):
    b = tw.shape[0]
    bpw = b // NW
    nchunks = bpw // CHUNK
    tw3 = tw.reshape(NW, nchunks, CHUNK).astype(jnp.int32)
    cw3 = cw.reshape(NW, nchunks, CHUNK).astype(jnp.int32)
    ng3 = ng.reshape(NW, nchunks, CHUNK).astype(jnp.int32)

    mesh = plsc.VectorSubcoreMesh(core_axis_name="c", subcore_axis_name="s")

    @functools.partial(
        pl.kernel,
        mesh=mesh,
        compiler_params=pltpu.CompilerParams(use_tc_tiling_on_sc=False),
        out_type=[
            jax.ShapeDtypeStruct((b, EMB), jnp.float32),
            jax.ShapeDtypeStruct((b, EMB), jnp.float32),
            jax.ShapeDtypeStruct((b, EMB), jnp.float32),
        ],
        scratch_types=[
            pltpu.VMEM((nchunks, CHUNK), jnp.int32),
            pltpu.VMEM((nchunks, CHUNK), jnp.int32),
            pltpu.VMEM((nchunks, CHUNK), jnp.int32),
            pltpu.VMEM((bpw, EMB), jnp.float32),
            pltpu.VMEM((bpw, EMB), jnp.float32),
            pltpu.VMEM((bpw, EMB), jnp.float32),
            pltpu.SemaphoreType.DMA,
        ],
    )
    def k(tw_hbm, cw_hbm, ng_hbm, temb_hbm, cemb_hbm,
          t_out, c_out, n_out,
          ti, ci, ni, tr, cr, nr, sem):
        wid = lax.axis_index("s") * NC + lax.axis_index("c")
        base = wid * bpw
        pltpu.sync_copy(tw_hbm.at[wid], ti)
        pltpu.sync_copy(cw_hbm.at[wid], ci)
        pltpu.sync_copy(ng_hbm.at[wid], ni)
        copies = []
        for j in range(nchunks):
            dst = pl.ds(j * CHUNK, CHUNK)
            copies.append(pltpu.async_copy(temb_hbm.at[ti.at[j]], tr.at[dst], sem))
            copies.append(pltpu.async_copy(cemb_hbm.at[ci.at[j]], cr.at[dst], sem))
            copies.append(pltpu.async_copy(temb_hbm.at[ni.at[j]], nr.at[dst], sem))
        for c in copies:
            c.wait()
        pltpu.sync_copy(tr, t_out.at[pl.ds(base, bpw)])
        pltpu.sync_copy(cr, c_out.at[pl.ds(base, bpw)])
        pltpu.sync_copy(nr, n_out.at[pl.ds(base, bpw)])

    return k(tw3, cw3, ng3, temb, cemb)


def _tc_loss(t_rows, c_rows, n_rows):
    def body(t_ref, c_ref, n_ref, o_ref):
        t = t_ref[...]
        c = c_ref[...]
        n = n_ref[...]
        pos = jnp.sum(t * c, axis=1)
        neg = jnp.sum(n * c, axis=1)
        def ls(x):
            return jnp.minimum(x, 0.0) - jnp.log1p(jnp.exp(-jnp.abs(x)))
        total = jnp.sum(ls(pos)) + jnp.sum(ls(-neg))
        o_ref[...] = jnp.full((1, 1), -total, jnp.float32)

    out = pl.pallas_call(
        body,
        out_shape=jax.ShapeDtypeStruct((1, 1), jnp.float32),
    )(t_rows, c_rows, n_rows)
    return out[0, 0]


def kernel(target_word, context_word, negative_example, target_emb, context_emb):
    t_rows, c_rows, n_rows = _sc_gather3(
        target_word, context_word, negative_example, target_emb, context_emb)
    return _tc_loss(t_rows, c_rows, n_rows)
